# Initial kernel scaffold; baseline (speedup 1.0000x reference)
import jax
import jax.numpy as jnp
from jax.experimental import pallas as pl


def kernel(x, edge_index, W1, b1, W2, b2, W3, b3):
    raise NotImplementedError("write your pallas kernel here")


# speedup vs baseline: 19.7576x; 19.7576x over previous
import functools

import jax
import jax.numpy as jnp
from jax import lax
from jax.experimental import pallas as pl
from jax.experimental.pallas import tpu as pltpu
from jax.experimental.pallas import tpu_sc as plsc

NC = 2
NS = 16
NW = NC * NS

N = 10000
D = 128
E = 320000
NP = 10240
CH = 128
RPW = 80
EP = NW * RPW * CH

RB = 2000
GRID = N // RB

_mesh = functools.partial(
    plsc.VectorSubcoreMesh, core_axis_name="core", subcore_axis_name="subcore"
)


def _zero_vec16():
    return jnp.zeros((16,), jnp.float32)


def _sc_degree(dst2d):
    @functools.partial(
        pl.kernel,
        out_type=jax.ShapeDtypeStruct((NC, NP), jnp.float32),
        mesh=_mesh(),
        scratch_types=[
            pltpu.VMEM((RPW, CH), jnp.int32),
            pltpu.VMEM((CH,), jnp.float32),
            pltpu.VMEM((NP // NS,), jnp.float32),
            pltpu.VMEM_SHARED((NP,), jnp.float32),
        ],
    )
    def deg_kernel(dst_hbm, degp_hbm, dst_v, ones_v, zbuf_v, deg_sh):
        c = lax.axis_index("core")
        s = lax.axis_index("subcore")
        w = c * NS + s
        nz = NP // NS

        @pl.loop(0, nz // 16)
        def _(i):
            zbuf_v[pl.ds(i * 16, 16)] = _zero_vec16()

        @pl.loop(0, CH // 16)
        def _(i):
            ones_v[pl.ds(i * 16, 16)] = jnp.full((16,), 1.0, jnp.float32)

        pltpu.sync_copy(zbuf_v, deg_sh.at[pl.ds(s * nz, nz)])
        pltpu.sync_copy(dst_hbm.at[pl.ds(w * RPW, RPW)], dst_v)
        plsc.subcore_barrier()

        @pl.loop(0, RPW)
        def _(j):
            pltpu.sync_copy(ones_v, deg_sh.at[dst_v.at[j]], add=True)

        plsc.subcore_barrier()
        pltpu.sync_copy(deg_sh.at[pl.ds(s * nz, nz)],
                        degp_hbm.at[c, pl.ds(s * nz, nz)])

    return deg_kernel(dst2d)


def _sc_scatter(g, src2d, dst2d):
    @functools.partial(
        pl.kernel,
        out_type=jax.ShapeDtypeStruct((NC, NP, D), jnp.float32),
        mesh=_mesh(),
        scratch_types=[
            pltpu.VMEM((RPW, CH), jnp.int32),
            pltpu.VMEM((RPW, CH), jnp.int32),
            pltpu.VMEM((CH, D), jnp.float32),
            pltpu.VMEM_SHARED((NP, D), jnp.float32),
        ],
    )
    def scat_kernel(g_hbm, src_hbm, dst_hbm, aggp_hbm, src_v, dst_v, rows_v, acc_sh):
        c = lax.axis_index("core")
        s = lax.axis_index("subcore")
        w = c * NS + s
        nz = NP // NS

        @pl.loop(0, CH)
        def _(r):
            @pl.loop(0, D // 16)
            def _(k):
                rows_v[r, pl.ds(k * 16, 16)] = _zero_vec16()

        for t in range(nz // CH):
            pltpu.sync_copy(rows_v, acc_sh.at[pl.ds(s * nz + t * CH, CH)])

        pltpu.sync_copy(src_hbm.at[pl.ds(w * RPW, RPW)], src_v)
        pltpu.sync_copy(dst_hbm.at[pl.ds(w * RPW, RPW)], dst_v)
        plsc.subcore_barrier()

        @pl.loop(0, RPW)
        def _(j):
            pltpu.sync_copy(g_hbm.at[src_v.at[j]], rows_v)
            pltpu.sync_copy(rows_v, acc_sh.at[dst_v.at[j]], add=True)

        plsc.subcore_barrier()
        for t in range(nz // CH):
            sl = pl.ds(s * nz + t * CH, CH)
            pltpu.sync_copy(acc_sh.at[sl], aggp_hbm.at[c, sl])

    return scat_kernel(g, src2d, dst2d)


def _tc_matmul(x, W):
    def body(x_ref, w_ref, o_ref):
        o_ref[...] = jnp.dot(x_ref[...], w_ref[...],
                             preferred_element_type=jnp.float32)

    return pl.pallas_call(
        body,
        grid=(GRID,),
        in_specs=[
            pl.BlockSpec((RB, D), lambda i: (i, 0)),
            pl.BlockSpec((D, D), lambda i: (0, 0)),
        ],
        out_specs=pl.BlockSpec((RB, D), lambda i: (i, 0)),
        out_shape=jax.ShapeDtypeStruct((N, D), jnp.float32),
    )(x, W)


def _dinv_block(degp_ref):
    d = degp_ref[...]
    return lax.rsqrt(d[0] + d[1] + 1.0)


def _tc_scale(h, degp3):
    def body(h_ref, degp_ref, o_ref):
        o_ref[...] = h_ref[...] * _dinv_block(degp_ref)

    return pl.pallas_call(
        body,
        grid=(GRID,),
        in_specs=[
            pl.BlockSpec((RB, D), lambda i: (i, 0)),
            pl.BlockSpec((NC, RB, 1), lambda i: (0, i, 0)),
        ],
        out_specs=pl.BlockSpec((RB, D), lambda i: (i, 0)),
        out_shape=jax.ShapeDtypeStruct((N, D), jnp.float32),
    )(h, degp3)


def _tc_mid(aggp, g, degp3, b, Wn):
    def body(a_ref, g_ref, degp_ref, b_ref, w_ref, o_ref):
        dinv = _dinv_block(degp_ref)
        a = a_ref[...]
        act = jnp.maximum(dinv * (a[0] + a[1] + g_ref[...]) + b_ref[...], 0.0)
        o_ref[...] = dinv * jnp.dot(act, w_ref[...],
                                    preferred_element_type=jnp.float32)

    return pl.pallas_call(
        body,
        grid=(GRID,),
        in_specs=[
            pl.BlockSpec((NC, RB, D), lambda i: (0, i, 0)),
            pl.BlockSpec((RB, D), lambda i: (i, 0)),
            pl.BlockSpec((NC, RB, 1), lambda i: (0, i, 0)),
            pl.BlockSpec((1, D), lambda i: (0, 0)),
            pl.BlockSpec((D, D), lambda i: (0, 0)),
        ],
        out_specs=pl.BlockSpec((RB, D), lambda i: (i, 0)),
        out_shape=jax.ShapeDtypeStruct((N, D), jnp.float32),
    )(aggp, g, degp3, b, Wn)


def _tc_final(aggp, g, degp3, b):
    def body(a_ref, g_ref, degp_ref, b_ref, o_ref):
        dinv = _dinv_block(degp_ref)
        a = a_ref[...]
        o_ref[...] = dinv * (a[0] + a[1] + g_ref[...]) + b_ref[...]

    return pl.pallas_call(
        body,
        grid=(GRID,),
        in_specs=[
            pl.BlockSpec((NC, RB, D), lambda i: (0, i, 0)),
            pl.BlockSpec((RB, D), lambda i: (i, 0)),
            pl.BlockSpec((NC, RB, 1), lambda i: (0, i, 0)),
            pl.BlockSpec((1, D), lambda i: (0, 0)),
        ],
        out_specs=pl.BlockSpec((RB, D), lambda i: (i, 0)),
        out_shape=jax.ShapeDtypeStruct((N, D), jnp.float32),
    )(aggp, g, degp3, b)


def kernel(x, edge_index, W1, b1, W2, b2, W3, b3):
    src = edge_index[0]
    dst = edge_index[1]

    npad = EP - E
    pad_ar = jnp.arange(npad, dtype=jnp.int32)
    src_p = jnp.concatenate([src, pad_ar % N])
    dst_p = jnp.concatenate([dst, N + pad_ar % (NP - N)])
    src2d = src_p.reshape(EP // CH, CH)
    dst2d = dst_p.reshape(EP // CH, CH)

    degp = _sc_degree(dst2d)
    degp3 = degp.reshape(NC, NP, 1)

    h1 = _tc_matmul(x, W1)
    g1 = _tc_scale(h1, degp3)
    a1 = _sc_scatter(g1, src2d, dst2d)
    g2 = _tc_mid(a1, g1, degp3, b1.reshape(1, D), W2)
    a2 = _sc_scatter(g2, src2d, dst2d)
    g3 = _tc_mid(a2, g2, degp3, b2.reshape(1, D), W3)
    a3 = _sc_scatter(g3, src2d, dst2d)
    return _tc_final(a3, g3, degp3, b3.reshape(1, D))


# speedup vs baseline: 25.4212x; 1.2867x over previous
import functools

import jax
import jax.numpy as jnp
from jax import lax
from jax.experimental import pallas as pl
from jax.experimental.pallas import tpu as pltpu
from jax.experimental.pallas import tpu_sc as plsc

NC = 2
NS = 16
NW = NC * NS

N = 10000
D = 128
E = 320000
NP = 10240
CH = 128
RPW = 80
EP = NW * RPW * CH

RB = 2000
GRID = N // RB

_mesh = functools.partial(
    plsc.VectorSubcoreMesh, core_axis_name="core", subcore_axis_name="subcore"
)


def _zero_vec16():
    return jnp.zeros((16,), jnp.float32)


def _sc_degree(dst2d):
    @functools.partial(
        pl.kernel,
        out_type=jax.ShapeDtypeStruct((NC, NP), jnp.float32),
        mesh=_mesh(),
        scratch_types=[
            pltpu.VMEM((RPW, CH), jnp.int32),
            pltpu.VMEM((CH,), jnp.float32),
            pltpu.VMEM((NP // NS,), jnp.float32),
            pltpu.VMEM_SHARED((NP,), jnp.float32),
        ],
    )
    def deg_kernel(dst_hbm, degp_hbm, dst_v, ones_v, zbuf_v, deg_sh):
        c = lax.axis_index("core")
        s = lax.axis_index("subcore")
        w = c * NS + s
        nz = NP // NS

        @pl.loop(0, nz // 16)
        def _(i):
            zbuf_v[pl.ds(i * 16, 16)] = _zero_vec16()

        @pl.loop(0, CH // 16)
        def _(i):
            ones_v[pl.ds(i * 16, 16)] = jnp.full((16,), 1.0, jnp.float32)

        pltpu.sync_copy(zbuf_v, deg_sh.at[pl.ds(s * nz, nz)])
        pltpu.sync_copy(dst_hbm.at[pl.ds(w * RPW, RPW)], dst_v)
        plsc.subcore_barrier()

        @pl.loop(0, RPW)
        def _(j):
            pltpu.sync_copy(ones_v, deg_sh.at[dst_v.at[j]], add=True)

        plsc.subcore_barrier()
        pltpu.sync_copy(deg_sh.at[pl.ds(s * nz, nz)],
                        degp_hbm.at[c, pl.ds(s * nz, nz)])

    return deg_kernel(dst2d)


def _sc_scatter(g, src2d, dst2d):
    @functools.partial(
        pl.kernel,
        out_type=jax.ShapeDtypeStruct((NC, NP, D), jnp.float32),
        mesh=_mesh(),
        scratch_types=[
            pltpu.VMEM((RPW // 2, CH), jnp.int32),
            pltpu.VMEM((RPW // 2, CH), jnp.int32),
            pltpu.VMEM((2, CH, D), jnp.float32),
            pltpu.VMEM_SHARED((NP, D), jnp.float32),
            pltpu.SemaphoreType.DMA,
            pltpu.SemaphoreType.DMA,
        ],
    )
    def scat_kernel(g_hbm, src_hbm, dst_hbm, aggp_hbm, src_v, dst_v, rows_v,
                    acc_sh, sem0, sem1):
        c = lax.axis_index("core")
        s = lax.axis_index("subcore")
        w = c * NS + s
        nz = NP // NS
        HR = RPW // 2
        b0 = rows_v.at[0]
        b1 = rows_v.at[1]

        @pl.loop(0, CH)
        def _(r):
            @pl.loop(0, D // 16)
            def _(k):
                rows_v[0, r, pl.ds(k * 16, 16)] = _zero_vec16()

        for t in range(nz // CH):
            pltpu.sync_copy(b0, acc_sh.at[pl.ds(s * nz + t * CH, CH)])

        plsc.subcore_barrier()

        for h in range(2):
            pltpu.sync_copy(src_hbm.at[pl.ds(w * RPW + h * HR, HR)], src_v)
            pltpu.sync_copy(dst_hbm.at[pl.ds(w * RPW + h * HR, HR)], dst_v)
            pltpu.async_copy(g_hbm.at[src_v.at[0]], b0, sem0)

            @pl.loop(0, HR, step=2)
            def _(j):
                pltpu.make_async_copy(g_hbm.at[src_v.at[j]], b0, sem0).wait()
                pltpu.async_copy(g_hbm.at[src_v.at[j + 1]], b1, sem1)
                pltpu.sync_copy(b0, acc_sh.at[dst_v.at[j]], add=True)
                pltpu.make_async_copy(g_hbm.at[src_v.at[j + 1]], b1, sem1).wait()

                @pl.when(j + 2 < HR)
                def _():
                    pltpu.async_copy(g_hbm.at[src_v.at[j + 2]], b0, sem0)

                pltpu.sync_copy(b1, acc_sh.at[dst_v.at[j + 1]], add=True)

        plsc.subcore_barrier()
        for t in range(nz // CH):
            sl = pl.ds(s * nz + t * CH, CH)
            pltpu.sync_copy(acc_sh.at[sl], aggp_hbm.at[c, sl])

    return scat_kernel(g, src2d, dst2d)


def _tc_matmul(x, W):
    def body(x_ref, w_ref, o_ref):
        o_ref[...] = jnp.dot(x_ref[...], w_ref[...],
                             preferred_element_type=jnp.float32)

    return pl.pallas_call(
        body,
        grid=(GRID,),
        in_specs=[
            pl.BlockSpec((RB, D), lambda i: (i, 0)),
            pl.BlockSpec((D, D), lambda i: (0, 0)),
        ],
        out_specs=pl.BlockSpec((RB, D), lambda i: (i, 0)),
        out_shape=jax.ShapeDtypeStruct((N, D), jnp.float32),
    )(x, W)


def _dinv_block(degp_ref):
    d = degp_ref[...]
    return lax.rsqrt(d[0] + d[1] + 1.0)


def _tc_scale(h, degp3):
    def body(h_ref, degp_ref, o_ref):
        o_ref[...] = h_ref[...] * _dinv_block(degp_ref)

    return pl.pallas_call(
        body,
        grid=(GRID,),
        in_specs=[
            pl.BlockSpec((RB, D), lambda i: (i, 0)),
            pl.BlockSpec((NC, RB, 1), lambda i: (0, i, 0)),
        ],
        out_specs=pl.BlockSpec((RB, D), lambda i: (i, 0)),
        out_shape=jax.ShapeDtypeStruct((N, D), jnp.float32),
    )(h, degp3)


def _tc_mid(aggp, g, degp3, b, Wn):
    def body(a_ref, g_ref, degp_ref, b_ref, w_ref, o_ref):
        dinv = _dinv_block(degp_ref)
        a = a_ref[...]
        act = jnp.maximum(dinv * (a[0] + a[1] + g_ref[...]) + b_ref[...], 0.0)
        o_ref[...] = dinv * jnp.dot(act, w_ref[...],
                                    preferred_element_type=jnp.float32)

    return pl.pallas_call(
        body,
        grid=(GRID,),
        in_specs=[
            pl.BlockSpec((NC, RB, D), lambda i: (0, i, 0)),
            pl.BlockSpec((RB, D), lambda i: (i, 0)),
            pl.BlockSpec((NC, RB, 1), lambda i: (0, i, 0)),
            pl.BlockSpec((1, D), lambda i: (0, 0)),
            pl.BlockSpec((D, D), lambda i: (0, 0)),
        ],
        out_specs=pl.BlockSpec((RB, D), lambda i: (i, 0)),
        out_shape=jax.ShapeDtypeStruct((N, D), jnp.float32),
    )(aggp, g, degp3, b, Wn)


def _tc_final(aggp, g, degp3, b):
    def body(a_ref, g_ref, degp_ref, b_ref, o_ref):
        dinv = _dinv_block(degp_ref)
        a = a_ref[...]
        o_ref[...] = dinv * (a[0] + a[1] + g_ref[...]) + b_ref[...]

    return pl.pallas_call(
        body,
        grid=(GRID,),
        in_specs=[
            pl.BlockSpec((NC, RB, D), lambda i: (0, i, 0)),
            pl.BlockSpec((RB, D), lambda i: (i, 0)),
            pl.BlockSpec((NC, RB, 1), lambda i: (0, i, 0)),
            pl.BlockSpec((1, D), lambda i: (0, 0)),
        ],
        out_specs=pl.BlockSpec((RB, D), lambda i: (i, 0)),
        out_shape=jax.ShapeDtypeStruct((N, D), jnp.float32),
    )(aggp, g, degp3, b)


def kernel(x, edge_index, W1, b1, W2, b2, W3, b3):
    src = edge_index[0]
    dst = edge_index[1]

    npad = EP - E
    pad_ar = jnp.arange(npad, dtype=jnp.int32)
    src_p = jnp.concatenate([src, pad_ar % N])
    dst_p = jnp.concatenate([dst, N + pad_ar % (NP - N)])
    src2d = src_p.reshape(EP // CH, CH)
    dst2d = dst_p.reshape(EP // CH, CH)

    degp = _sc_degree(dst2d)
    degp3 = degp.reshape(NC, NP, 1)

    h1 = _tc_matmul(x, W1)
    g1 = _tc_scale(h1, degp3)
    a1 = _sc_scatter(g1, src2d, dst2d)
    g2 = _tc_mid(a1, g1, degp3, b1.reshape(1, D), W2)
    a2 = _sc_scatter(g2, src2d, dst2d)
    g3 = _tc_mid(a2, g2, degp3, b2.reshape(1, D), W3)
    a3 = _sc_scatter(g3, src2d, dst2d)
    return _tc_final(a3, g3, degp3, b3.reshape(1, D))


# speedup vs baseline: 30.2836x; 1.1913x over previous
import functools

import jax
import jax.numpy as jnp
from jax import lax
from jax.experimental import pallas as pl
from jax.experimental.pallas import tpu as pltpu
from jax.experimental.pallas import tpu_sc as plsc

NC = 2
NS = 16
NW = NC * NS

N = 10000
D = 128
E = 320000
NP = 10240
CH = 64
RPW = 160
EP = NW * RPW * CH
NBUF = 4

RB = 2000
GRID = N // RB

_mesh = functools.partial(
    plsc.VectorSubcoreMesh, core_axis_name="core", subcore_axis_name="subcore"
)


def _zero_vec16():
    return jnp.zeros((16,), jnp.float32)


def _sc_degree(dst2d):
    @functools.partial(
        pl.kernel,
        out_type=jax.ShapeDtypeStruct((NC, NP), jnp.float32),
        mesh=_mesh(),
        scratch_types=[
            pltpu.VMEM((RPW, CH), jnp.int32),
            pltpu.VMEM((CH,), jnp.float32),
            pltpu.VMEM((NP // NS,), jnp.float32),
            pltpu.VMEM_SHARED((NP,), jnp.float32),
        ],
    )
    def deg_kernel(dst_hbm, degp_hbm, dst_v, ones_v, zbuf_v, deg_sh):
        c = lax.axis_index("core")
        s = lax.axis_index("subcore")
        w = c * NS + s
        nz = NP // NS

        @pl.loop(0, nz // 16)
        def _(i):
            zbuf_v[pl.ds(i * 16, 16)] = _zero_vec16()

        @pl.loop(0, CH // 16)
        def _(i):
            ones_v[pl.ds(i * 16, 16)] = jnp.full((16,), 1.0, jnp.float32)

        pltpu.sync_copy(zbuf_v, deg_sh.at[pl.ds(s * nz, nz)])
        pltpu.sync_copy(dst_hbm.at[pl.ds(w * RPW, RPW)], dst_v)
        plsc.subcore_barrier()

        @pl.loop(0, RPW)
        def _(j):
            pltpu.sync_copy(ones_v, deg_sh.at[dst_v.at[j]], add=True)

        plsc.subcore_barrier()
        pltpu.sync_copy(deg_sh.at[pl.ds(s * nz, nz)],
                        degp_hbm.at[c, pl.ds(s * nz, nz)])

    return deg_kernel(dst2d)


def _sc_scatter(g, src2d, dst2d):
    @functools.partial(
        pl.kernel,
        out_type=jax.ShapeDtypeStruct((NC, NP, D), jnp.float32),
        mesh=_mesh(),
        scratch_types=[
            pltpu.VMEM((RPW // 4, CH), jnp.int32),
            pltpu.VMEM((RPW // 4, CH), jnp.int32),
            pltpu.VMEM((NBUF, CH, D), jnp.float32),
            pltpu.VMEM_SHARED((NP, D), jnp.float32),
            [pltpu.SemaphoreType.DMA] * NBUF,
        ],
    )
    def scat_kernel(g_hbm, src_hbm, dst_hbm, aggp_hbm, src_v, dst_v, rows_v,
                    acc_sh, sems):
        c = lax.axis_index("core")
        s = lax.axis_index("subcore")
        w = c * NS + s
        nz = NP // NS
        HR = RPW // 4
        bufs = [rows_v.at[k] for k in range(NBUF)]

        @pl.loop(0, CH)
        def _(r):
            @pl.loop(0, D // 16)
            def _(k):
                rows_v[0, r, pl.ds(k * 16, 16)] = _zero_vec16()

        for t in range(nz // CH):
            pltpu.sync_copy(bufs[0], acc_sh.at[pl.ds(s * nz + t * CH, CH)])

        plsc.subcore_barrier()

        for h in range(4):
            pltpu.sync_copy(src_hbm.at[pl.ds(w * RPW + h * HR, HR)], src_v)
            pltpu.sync_copy(dst_hbm.at[pl.ds(w * RPW + h * HR, HR)], dst_v)
            for k in range(NBUF - 1):
                pltpu.async_copy(g_hbm.at[src_v.at[k]], bufs[k], sems[k])

            @pl.loop(0, HR, step=NBUF)
            def _(j):
                for k in range(NBUF):
                    pltpu.make_async_copy(
                        g_hbm.at[src_v.at[j + k]], bufs[k], sems[k]).wait()
                    nxt = j + k + NBUF - 1
                    kn = (k + NBUF - 1) % NBUF

                    @pl.when(nxt < HR)
                    def _():
                        pltpu.async_copy(
                            g_hbm.at[src_v.at[nxt]], bufs[kn], sems[kn])

                    pltpu.sync_copy(bufs[k], acc_sh.at[dst_v.at[j + k]],
                                    add=True)

        plsc.subcore_barrier()
        for t in range(nz // CH):
            sl = pl.ds(s * nz + t * CH, CH)
            pltpu.sync_copy(acc_sh.at[sl], aggp_hbm.at[c, sl])

    return scat_kernel(g, src2d, dst2d)


def _tc_matmul(x, W):
    def body(x_ref, w_ref, o_ref):
        o_ref[...] = jnp.dot(x_ref[...], w_ref[...],
                             preferred_element_type=jnp.float32)

    return pl.pallas_call(
        body,
        grid=(GRID,),
        in_specs=[
            pl.BlockSpec((RB, D), lambda i: (i, 0)),
            pl.BlockSpec((D, D), lambda i: (0, 0)),
        ],
        out_specs=pl.BlockSpec((RB, D), lambda i: (i, 0)),
        out_shape=jax.ShapeDtypeStruct((N, D), jnp.float32),
    )(x, W)


def _dinv_block(degp_ref):
    d = degp_ref[...]
    return lax.rsqrt(d[0] + d[1] + 1.0)


def _tc_scale(h, degp3):
    def body(h_ref, degp_ref, o_ref):
        o_ref[...] = h_ref[...] * _dinv_block(degp_ref)

    return pl.pallas_call(
        body,
        grid=(GRID,),
        in_specs=[
            pl.BlockSpec((RB, D), lambda i: (i, 0)),
            pl.BlockSpec((NC, RB, 1), lambda i: (0, i, 0)),
        ],
        out_specs=pl.BlockSpec((RB, D), lambda i: (i, 0)),
        out_shape=jax.ShapeDtypeStruct((N, D), jnp.float32),
    )(h, degp3)


def _tc_mid(aggp, g, degp3, b, Wn):
    def body(a_ref, g_ref, degp_ref, b_ref, w_ref, o_ref):
        dinv = _dinv_block(degp_ref)
        a = a_ref[...]
        act = jnp.maximum(dinv * (a[0] + a[1] + g_ref[...]) + b_ref[...], 0.0)
        o_ref[...] = dinv * jnp.dot(act, w_ref[...],
                                    preferred_element_type=jnp.float32)

    return pl.pallas_call(
        body,
        grid=(GRID,),
        in_specs=[
            pl.BlockSpec((NC, RB, D), lambda i: (0, i, 0)),
            pl.BlockSpec((RB, D), lambda i: (i, 0)),
            pl.BlockSpec((NC, RB, 1), lambda i: (0, i, 0)),
            pl.BlockSpec((1, D), lambda i: (0, 0)),
            pl.BlockSpec((D, D), lambda i: (0, 0)),
        ],
        out_specs=pl.BlockSpec((RB, D), lambda i: (i, 0)),
        out_shape=jax.ShapeDtypeStruct((N, D), jnp.float32),
    )(aggp, g, degp3, b, Wn)


def _tc_final(aggp, g, degp3, b):
    def body(a_ref, g_ref, degp_ref, b_ref, o_ref):
        dinv = _dinv_block(degp_ref)
        a = a_ref[...]
        o_ref[...] = dinv * (a[0] + a[1] + g_ref[...]) + b_ref[...]

    return pl.pallas_call(
        body,
        grid=(GRID,),
        in_specs=[
            pl.BlockSpec((NC, RB, D), lambda i: (0, i, 0)),
            pl.BlockSpec((RB, D), lambda i: (i, 0)),
            pl.BlockSpec((NC, RB, 1), lambda i: (0, i, 0)),
            pl.BlockSpec((1, D), lambda i: (0, 0)),
        ],
        out_specs=pl.BlockSpec((RB, D), lambda i: (i, 0)),
        out_shape=jax.ShapeDtypeStruct((N, D), jnp.float32),
    )(aggp, g, degp3, b)


def kernel(x, edge_index, W1, b1, W2, b2, W3, b3):
    src = edge_index[0]
    dst = edge_index[1]

    npad = EP - E
    pad_ar = jnp.arange(npad, dtype=jnp.int32)
    src_p = jnp.concatenate([src, pad_ar % N])
    dst_p = jnp.concatenate([dst, N + pad_ar % (NP - N)])
    src2d = src_p.reshape(EP // CH, CH)
    dst2d = dst_p.reshape(EP // CH, CH)

    degp = _sc_degree(dst2d)
    degp3 = degp.reshape(NC, NP, 1)

    h1 = _tc_matmul(x, W1)
    g1 = _tc_scale(h1, degp3)
    a1 = _sc_scatter(g1, src2d, dst2d)
    g2 = _tc_mid(a1, g1, degp3, b1.reshape(1, D), W2)
    a2 = _sc_scatter(g2, src2d, dst2d)
    g3 = _tc_mid(a2, g2, degp3, b2.reshape(1, D), W3)
    a3 = _sc_scatter(g3, src2d, dst2d)
    return _tc_final(a3, g3, degp3, b3.reshape(1, D))


# speedup vs baseline: 30.5187x; 1.0078x over previous
import functools

import jax
import jax.numpy as jnp
from jax import lax
from jax.experimental import pallas as pl
from jax.experimental.pallas import tpu as pltpu
from jax.experimental.pallas import tpu_sc as plsc

NC = 2
NS = 16
NW = NC * NS

N = 10000
D = 128
E = 320000
NP = 10240
CH = 64
RPW = 160
EP = NW * RPW * CH
NBUF = 4
HRQ = 16

RB = 2000
GRID = N // RB

_mesh = functools.partial(
    plsc.VectorSubcoreMesh, core_axis_name="core", subcore_axis_name="subcore"
)


def _zero_vec16():
    return jnp.zeros((16,), jnp.float32)


def _sc_degree(dst2d):
    @functools.partial(
        pl.kernel,
        out_type=jax.ShapeDtypeStruct((NC, NP), jnp.float32),
        mesh=_mesh(),
        scratch_types=[
            pltpu.VMEM((RPW, CH), jnp.int32),
            pltpu.VMEM((CH,), jnp.float32),
            pltpu.VMEM((NP // NS,), jnp.float32),
            pltpu.VMEM_SHARED((NP,), jnp.float32),
        ],
    )
    def deg_kernel(dst_hbm, degp_hbm, dst_v, ones_v, zbuf_v, deg_sh):
        c = lax.axis_index("core")
        s = lax.axis_index("subcore")
        w = c * NS + s
        nz = NP // NS

        @pl.loop(0, nz // 16)
        def _(i):
            zbuf_v[pl.ds(i * 16, 16)] = _zero_vec16()

        @pl.loop(0, CH // 16)
        def _(i):
            ones_v[pl.ds(i * 16, 16)] = jnp.full((16,), 1.0, jnp.float32)

        pltpu.sync_copy(zbuf_v, deg_sh.at[pl.ds(s * nz, nz)])
        pltpu.sync_copy(dst_hbm.at[pl.ds(w * RPW, RPW)], dst_v)
        plsc.subcore_barrier()

        @pl.loop(0, RPW)
        def _(j):
            pltpu.sync_copy(ones_v, deg_sh.at[dst_v.at[j]], add=True)

        plsc.subcore_barrier()
        pltpu.sync_copy(deg_sh.at[pl.ds(s * nz, nz)],
                        degp_hbm.at[c, pl.ds(s * nz, nz)])

    return deg_kernel(dst2d)


def _sc_scatter(g, src2d, dst2d):
    @functools.partial(
        pl.kernel,
        out_type=jax.ShapeDtypeStruct((NC, NP, D), jnp.float32),
        mesh=_mesh(),
        scratch_types=[
            pltpu.VMEM((2, HRQ, CH), jnp.int32),
            pltpu.VMEM((2, HRQ, CH), jnp.int32),
            pltpu.VMEM((NBUF, CH, D), jnp.float32),
            pltpu.VMEM_SHARED((NP, D), jnp.float32),
            [pltpu.SemaphoreType.DMA] * NBUF,
        ],
    )
    def scat_kernel(g_hbm, src_hbm, dst_hbm, aggp_hbm, src_v, dst_v, rows_v,
                    acc_sh, sems):
        c = lax.axis_index("core")
        s = lax.axis_index("subcore")
        w = c * NS + s
        nz = NP // NS
        bufs = [rows_v.at[k] for k in range(NBUF)]

        @pl.loop(0, CH)
        def _(r):
            @pl.loop(0, D // 16)
            def _(k):
                rows_v[0, r, pl.ds(k * 16, 16)] = _zero_vec16()

        for t in range(nz // CH):
            pltpu.async_copy(bufs[0], acc_sh.at[pl.ds(s * nz + t * CH, CH)],
                             sems[0])
        for t in range(nz // CH):
            pltpu.make_async_copy(bufs[0], acc_sh.at[pl.ds(CH, CH)],
                                  sems[0]).wait()

        plsc.subcore_barrier()

        pltpu.sync_copy(src_hbm.at[pl.ds(w * RPW, HRQ)], src_v.at[0])
        pltpu.sync_copy(dst_hbm.at[pl.ds(w * RPW, HRQ)], dst_v.at[0])
        pltpu.sync_copy(src_hbm.at[pl.ds(w * RPW + HRQ, HRQ)], src_v.at[1])
        pltpu.sync_copy(dst_hbm.at[pl.ds(w * RPW + HRQ, HRQ)], dst_v.at[1])
        for k in range(NBUF - 1):
            pltpu.async_copy(g_hbm.at[src_v.at[0, k]], bufs[k], sems[k])

        @pl.loop(0, RPW, step=NBUF)
        def _(j):
            for k in range(NBUF):
                m = j + k
                pltpu.make_async_copy(
                    g_hbm.at[src_v.at[0, 0]], bufs[k], sems[k]).wait()
                q = m // HRQ

                @pl.when((m >= HRQ) & (m % HRQ == 0) & (q + 1 < RPW // HRQ))
                def _():
                    qb = (q + 1) % 2
                    off = pl.ds(w * RPW + (q + 1) * HRQ, HRQ)
                    pltpu.sync_copy(src_hbm.at[off], src_v.at[qb])
                    pltpu.sync_copy(dst_hbm.at[off], dst_v.at[qb])

                nxt = m + NBUF - 1
                kn = (k + NBUF - 1) % NBUF

                @pl.when(nxt < RPW)
                def _():
                    pltpu.async_copy(
                        g_hbm.at[src_v.at[(nxt // HRQ) % 2, nxt % HRQ]],
                        bufs[kn], sems[kn])

                pltpu.sync_copy(bufs[k], acc_sh.at[dst_v.at[q % 2, m % HRQ]],
                                add=True)

        plsc.subcore_barrier()
        sl = pl.ds(s * nz, nz)
        pltpu.sync_copy(acc_sh.at[sl], aggp_hbm.at[c, sl])

    return scat_kernel(g, src2d, dst2d)


def _tc_matmul(x, W):
    def body(x_ref, w_ref, o_ref):
        o_ref[...] = jnp.dot(x_ref[...], w_ref[...],
                             preferred_element_type=jnp.float32)

    return pl.pallas_call(
        body,
        grid=(GRID,),
        in_specs=[
            pl.BlockSpec((RB, D), lambda i: (i, 0)),
            pl.BlockSpec((D, D), lambda i: (0, 0)),
        ],
        out_specs=pl.BlockSpec((RB, D), lambda i: (i, 0)),
        out_shape=jax.ShapeDtypeStruct((N, D), jnp.float32),
    )(x, W)


def _dinv_block(degp_ref):
    d = degp_ref[...]
    return lax.rsqrt(d[0] + d[1] + 1.0)


def _tc_scale(h, degp3):
    def body(h_ref, degp_ref, o_ref):
        o_ref[...] = h_ref[...] * _dinv_block(degp_ref)

    return pl.pallas_call(
        body,
        grid=(GRID,),
        in_specs=[
            pl.BlockSpec((RB, D), lambda i: (i, 0)),
            pl.BlockSpec((NC, RB, 1), lambda i: (0, i, 0)),
        ],
        out_specs=pl.BlockSpec((RB, D), lambda i: (i, 0)),
        out_shape=jax.ShapeDtypeStruct((N, D), jnp.float32),
    )(h, degp3)


def _tc_mid(aggp, g, degp3, b, Wn):
    def body(a_ref, g_ref, degp_ref, b_ref, w_ref, o_ref):
        dinv = _dinv_block(degp_ref)
        a = a_ref[...]
        act = jnp.maximum(dinv * (a[0] + a[1] + g_ref[...]) + b_ref[...], 0.0)
        o_ref[...] = dinv * jnp.dot(act, w_ref[...],
                                    preferred_element_type=jnp.float32)

    return pl.pallas_call(
        body,
        grid=(GRID,),
        in_specs=[
            pl.BlockSpec((NC, RB, D), lambda i: (0, i, 0)),
            pl.BlockSpec((RB, D), lambda i: (i, 0)),
            pl.BlockSpec((NC, RB, 1), lambda i: (0, i, 0)),
            pl.BlockSpec((1, D), lambda i: (0, 0)),
            pl.BlockSpec((D, D), lambda i: (0, 0)),
        ],
        out_specs=pl.BlockSpec((RB, D), lambda i: (i, 0)),
        out_shape=jax.ShapeDtypeStruct((N, D), jnp.float32),
    )(aggp, g, degp3, b, Wn)


def _tc_final(aggp, g, degp3, b):
    def body(a_ref, g_ref, degp_ref, b_ref, o_ref):
        dinv = _dinv_block(degp_ref)
        a = a_ref[...]
        o_ref[...] = dinv * (a[0] + a[1] + g_ref[...]) + b_ref[...]

    return pl.pallas_call(
        body,
        grid=(GRID,),
        in_specs=[
            pl.BlockSpec((NC, RB, D), lambda i: (0, i, 0)),
            pl.BlockSpec((RB, D), lambda i: (i, 0)),
            pl.BlockSpec((NC, RB, 1), lambda i: (0, i, 0)),
            pl.BlockSpec((1, D), lambda i: (0, 0)),
        ],
        out_specs=pl.BlockSpec((RB, D), lambda i: (i, 0)),
        out_shape=jax.ShapeDtypeStruct((N, D), jnp.float32),
    )(aggp, g, degp3, b)


def kernel(x, edge_index, W1, b1, W2, b2, W3, b3):
    src = edge_index[0]
    dst = edge_index[1]

    npad = EP - E
    pad_ar = jnp.arange(npad, dtype=jnp.int32)
    src_p = jnp.concatenate([src, pad_ar % N])
    dst_p = jnp.concatenate([dst, N + pad_ar % (NP - N)])
    src2d = src_p.reshape(EP // CH, CH)
    dst2d = dst_p.reshape(EP // CH, CH)

    degp = _sc_degree(dst2d)
    degp3 = degp.reshape(NC, NP, 1)

    h1 = _tc_matmul(x, W1)
    g1 = _tc_scale(h1, degp3)
    a1 = _sc_scatter(g1, src2d, dst2d)
    g2 = _tc_mid(a1, g1, degp3, b1.reshape(1, D), W2)
    a2 = _sc_scatter(g2, src2d, dst2d)
    g3 = _tc_mid(a2, g2, degp3, b2.reshape(1, D), W3)
    a3 = _sc_scatter(g3, src2d, dst2d)
    return _tc_final(a3, g3, degp3, b3.reshape(1, D))


# speedup vs baseline: 31.4631x; 1.0309x over previous
import functools

import jax
import jax.numpy as jnp
from jax import lax
from jax.experimental import pallas as pl
from jax.experimental.pallas import tpu as pltpu
from jax.experimental.pallas import tpu_sc as plsc

NC = 2
NS = 16
NW = NC * NS

N = 10000
D = 128
E = 320000
NP = 10240
CH = 80
RPW = 128
EP = NW * RPW * CH
NBUF = 4
HRQ = 16
DCH = 128
DRP = EP // NW // DCH

RB = 2000
GRID = N // RB

_mesh = functools.partial(
    plsc.VectorSubcoreMesh, core_axis_name="core", subcore_axis_name="subcore"
)


def _zero_vec16():
    return jnp.zeros((16,), jnp.float32)


def _sc_degree(dst2d):
    @functools.partial(
        pl.kernel,
        out_type=jax.ShapeDtypeStruct((NC, NP), jnp.float32),
        mesh=_mesh(),
        scratch_types=[
            pltpu.VMEM((DRP, DCH), jnp.int32),
            pltpu.VMEM((DCH,), jnp.float32),
            pltpu.VMEM((NP // NS,), jnp.float32),
            pltpu.VMEM_SHARED((NP,), jnp.float32),
        ],
    )
    def deg_kernel(dst_hbm, degp_hbm, dst_v, ones_v, zbuf_v, deg_sh):
        c = lax.axis_index("core")
        s = lax.axis_index("subcore")
        w = c * NS + s
        nz = NP // NS

        @pl.loop(0, nz // 16)
        def _(i):
            zbuf_v[pl.ds(i * 16, 16)] = _zero_vec16()

        @pl.loop(0, DCH // 16)
        def _(i):
            ones_v[pl.ds(i * 16, 16)] = jnp.full((16,), 1.0, jnp.float32)

        pltpu.sync_copy(zbuf_v, deg_sh.at[pl.ds(s * nz, nz)])
        pltpu.sync_copy(dst_hbm.at[pl.ds(w * DRP, DRP)], dst_v)
        plsc.subcore_barrier()

        @pl.loop(0, DRP)
        def _(j):
            pltpu.sync_copy(ones_v, deg_sh.at[dst_v.at[j]], add=True)

        plsc.subcore_barrier()
        pltpu.sync_copy(deg_sh.at[pl.ds(s * nz, nz)],
                        degp_hbm.at[c, pl.ds(s * nz, nz)])

    return deg_kernel(dst2d)


def _sc_scatter(g, src2d, dst2d):
    @functools.partial(
        pl.kernel,
        out_type=jax.ShapeDtypeStruct((NC, NP, D), jnp.float32),
        mesh=_mesh(),
        scratch_types=[
            pltpu.VMEM((2, HRQ, CH), jnp.int32),
            pltpu.VMEM((2, HRQ, CH), jnp.int32),
            pltpu.VMEM((NBUF, CH, D), jnp.float32),
            pltpu.VMEM_SHARED((NP, D), jnp.float32),
            [pltpu.SemaphoreType.DMA] * NBUF,
        ],
    )
    def scat_kernel(g_hbm, src_hbm, dst_hbm, aggp_hbm, src_v, dst_v, rows_v,
                    acc_sh, sems):
        c = lax.axis_index("core")
        s = lax.axis_index("subcore")
        w = c * NS + s
        nz = NP // NS
        bufs = [rows_v.at[k] for k in range(NBUF)]

        @pl.loop(0, CH)
        def _(r):
            @pl.loop(0, D // 16)
            def _(k):
                rows_v[0, r, pl.ds(k * 16, 16)] = _zero_vec16()

        for t in range(nz // CH):
            pltpu.async_copy(bufs[0], acc_sh.at[pl.ds(s * nz + t * CH, CH)],
                             sems[0])
        for t in range(nz // CH):
            pltpu.make_async_copy(bufs[0], acc_sh.at[pl.ds(CH, CH)],
                                  sems[0]).wait()

        plsc.subcore_barrier()

        pltpu.sync_copy(src_hbm.at[pl.ds(w * RPW, HRQ)], src_v.at[0])
        pltpu.sync_copy(dst_hbm.at[pl.ds(w * RPW, HRQ)], dst_v.at[0])
        pltpu.sync_copy(src_hbm.at[pl.ds(w * RPW + HRQ, HRQ)], src_v.at[1])
        pltpu.sync_copy(dst_hbm.at[pl.ds(w * RPW + HRQ, HRQ)], dst_v.at[1])
        for k in range(NBUF - 1):
            pltpu.async_copy(g_hbm.at[src_v.at[0, k]], bufs[k], sems[k])

        @pl.loop(0, RPW, step=NBUF)
        def _(j):
            for k in range(NBUF):
                m = j + k
                pltpu.make_async_copy(
                    g_hbm.at[src_v.at[0, 0]], bufs[k], sems[k]).wait()
                q = m // HRQ

                @pl.when((m >= HRQ) & (m % HRQ == 0) & (q + 1 < RPW // HRQ))
                def _():
                    qb = (q + 1) % 2
                    off = pl.ds(w * RPW + (q + 1) * HRQ, HRQ)
                    pltpu.sync_copy(src_hbm.at[off], src_v.at[qb])
                    pltpu.sync_copy(dst_hbm.at[off], dst_v.at[qb])

                nxt = m + NBUF - 1
                kn = (k + NBUF - 1) % NBUF

                @pl.when(nxt < RPW)
                def _():
                    pltpu.async_copy(
                        g_hbm.at[src_v.at[(nxt // HRQ) % 2, nxt % HRQ]],
                        bufs[kn], sems[kn])

                pltpu.sync_copy(bufs[k], acc_sh.at[dst_v.at[q % 2, m % HRQ]],
                                add=True)

        plsc.subcore_barrier()
        sl = pl.ds(s * nz, nz)
        pltpu.sync_copy(acc_sh.at[sl], aggp_hbm.at[c, sl])

    return scat_kernel(g, src2d, dst2d)


def _tc_pro(x, W, degp3):
    def body(x_ref, w_ref, degp_ref, o_ref):
        o_ref[...] = _dinv_block(degp_ref) * jnp.dot(
            x_ref[...], w_ref[...], preferred_element_type=jnp.float32)

    return pl.pallas_call(
        body,
        grid=(GRID,),
        in_specs=[
            pl.BlockSpec((RB, D), lambda i: (i, 0)),
            pl.BlockSpec((D, D), lambda i: (0, 0)),
            pl.BlockSpec((NC, RB, 1), lambda i: (0, i, 0)),
        ],
        out_specs=pl.BlockSpec((RB, D), lambda i: (i, 0)),
        out_shape=jax.ShapeDtypeStruct((N, D), jnp.float32),
    )(x, W, degp3)


def _dinv_block(degp_ref):
    d = degp_ref[...]
    return lax.rsqrt(d[0] + d[1] + 1.0)


def _tc_mid(aggp, g, degp3, b, Wn):
    def body(a_ref, g_ref, degp_ref, b_ref, w_ref, o_ref):
        dinv = _dinv_block(degp_ref)
        a = a_ref[...]
        act = jnp.maximum(dinv * (a[0] + a[1] + g_ref[...]) + b_ref[...], 0.0)
        o_ref[...] = dinv * jnp.dot(act, w_ref[...],
                                    preferred_element_type=jnp.float32)

    return pl.pallas_call(
        body,
        grid=(GRID,),
        in_specs=[
            pl.BlockSpec((NC, RB, D), lambda i: (0, i, 0)),
            pl.BlockSpec((RB, D), lambda i: (i, 0)),
            pl.BlockSpec((NC, RB, 1), lambda i: (0, i, 0)),
            pl.BlockSpec((1, D), lambda i: (0, 0)),
            pl.BlockSpec((D, D), lambda i: (0, 0)),
        ],
        out_specs=pl.BlockSpec((RB, D), lambda i: (i, 0)),
        out_shape=jax.ShapeDtypeStruct((N, D), jnp.float32),
    )(aggp, g, degp3, b, Wn)


def _tc_final(aggp, g, degp3, b):
    def body(a_ref, g_ref, degp_ref, b_ref, o_ref):
        dinv = _dinv_block(degp_ref)
        a = a_ref[...]
        o_ref[...] = dinv * (a[0] + a[1] + g_ref[...]) + b_ref[...]

    return pl.pallas_call(
        body,
        grid=(GRID,),
        in_specs=[
            pl.BlockSpec((NC, RB, D), lambda i: (0, i, 0)),
            pl.BlockSpec((RB, D), lambda i: (i, 0)),
            pl.BlockSpec((NC, RB, 1), lambda i: (0, i, 0)),
            pl.BlockSpec((1, D), lambda i: (0, 0)),
        ],
        out_specs=pl.BlockSpec((RB, D), lambda i: (i, 0)),
        out_shape=jax.ShapeDtypeStruct((N, D), jnp.float32),
    )(aggp, g, degp3, b)


def kernel(x, edge_index, W1, b1, W2, b2, W3, b3):
    src = edge_index[0]
    dst = edge_index[1]

    npad = EP - E
    pad_ar = jnp.arange(npad, dtype=jnp.int32)
    src_p = jnp.concatenate([src, pad_ar % N])
    dst_p = jnp.concatenate([dst, N + pad_ar % (NP - N)])
    src2d = src_p.reshape(EP // CH, CH)
    dst2d = dst_p.reshape(EP // CH, CH)

    degp = _sc_degree(dst_p.reshape(EP // DCH, DCH))
    degp3 = degp.reshape(NC, NP, 1)

    g1 = _tc_pro(x, W1, degp3)
    a1 = _sc_scatter(g1, src2d, dst2d)
    g2 = _tc_mid(a1, g1, degp3, b1.reshape(1, D), W2)
    a2 = _sc_scatter(g2, src2d, dst2d)
    g3 = _tc_mid(a2, g2, degp3, b2.reshape(1, D), W3)
    a3 = _sc_scatter(g3, src2d, dst2d)
    return _tc_final(a3, g3, degp3, b3.reshape(1, D))
